# Initial kernel scaffold; baseline (speedup 1.0000x reference)
import jax
import jax.numpy as jnp
from jax import lax
from jax.experimental import pallas as pl

_K = 8
_HI = lax.Precision.HIGHEST


def _body(x_ref, wq_ref, wk_ref, wv_ref, wo_ref, w2_ref, b_ref, out_ref):
    c, n = x_ref.shape[1], x_ref.shape[2]
    xb = x_ref[0]

    def nt_dot(a, b, prec):
        return lax.dot_general(a, b, (((1,), (1,)), ((), ())),
                               precision=prec,
                               preferred_element_type=jnp.float32)

    q = nt_dot(xb, wq_ref[...], _HI)
    k = nt_dot(xb, wk_ref[...], _HI)
    v = nt_dot(xb, wv_ref[...], _HI)

    ks = jnp.sqrt(jnp.sum(k * k, axis=0, keepdims=True))
    kn = k / jnp.maximum(ks, 1e-12)
    qs = jnp.sqrt(jnp.sum(q * q, axis=0, keepdims=True))
    qn = q / jnp.maximum(qs, 1e-12)

    sim = lax.dot_general(kn, qn, (((0,), (0,)), ((), ())),
                          precision=_HI, preferred_element_type=jnp.float32)
    sim = jnp.maximum(sim, 0.0)

    iota_m = lax.broadcasted_iota(jnp.int32, (n, n), 1)
    vhi = v.astype(jnp.bfloat16)
    vlo = (v - vhi.astype(jnp.float32)).astype(jnp.bfloat16)

    xc = jnp.zeros((c, n), jnp.float32)
    for kk in range(_K):
        mx = jnp.max(sim, axis=1, keepdims=True)
        am = jnp.min(jnp.where(sim == mx, iota_m, n), axis=1,
                     keepdims=True)
        sim = jnp.where(iota_m == am, -1.0, sim)
        pt = (iota_m == am).astype(jnp.bfloat16)
        g = nt_dot(vhi, pt, None) + nt_dot(vlo, pt, None)
        xc = xc + lax.dot_general(w2_ref[kk], g, (((1,), (0,)), ((), ())),
                                  precision=_HI,
                                  preferred_element_type=jnp.float32)
    xc = xc + b_ref[...]
    out_ref[0] = nt_dot(xc, wo_ref[...], _HI)


def kernel(x, Wq, Wk, Wv, Wo, conv_w, conv_b):
    b, c, h, w = x.shape
    n = h * w
    xf = x.reshape(b, c, n)
    w2 = conv_w.transpose(2, 0, 1)
    bias = conv_b.reshape(c, 1)

    full = lambda shp: pl.BlockSpec(shp, lambda i: tuple(0 for _ in shp))
    out = pl.pallas_call(
        _body,
        grid=(b,),
        in_specs=[
            pl.BlockSpec((1, c, n), lambda i: (i, 0, 0)),
            full((n, n)), full((n, n)), full((n, n)), full((n, n)),
            full((_K, c, c)),
            full((c, 1)),
        ],
        out_specs=pl.BlockSpec((1, c, n), lambda i: (i, 0, 0)),
        out_shape=jax.ShapeDtypeStruct((b, c, n), jnp.float32),
    )(xf, Wq, Wk, Wv, Wo, w2, bias)
    return out.reshape(b, c, h, w)


# speedup vs baseline: 9.1781x; 9.1781x over previous
import jax
import jax.numpy as jnp
from jax import lax
from jax.experimental import pallas as pl

_K = 8
_HI = lax.Precision.HIGHEST


def _body(x_ref, wq_ref, wk_ref, wv_ref, wo_ref, w2_ref, b_ref, out_ref):
    c, n = x_ref.shape[1], x_ref.shape[2]
    xb = x_ref[0]

    def nt_dot(a, b, prec):
        return lax.dot_general(a, b, (((1,), (1,)), ((), ())),
                               precision=prec,
                               preferred_element_type=jnp.float32)

    q = nt_dot(xb, wq_ref[...], None)
    k = nt_dot(xb, wk_ref[...], None)
    v = nt_dot(xb, wv_ref[...], _HI)

    ks = jnp.sqrt(jnp.sum(k * k, axis=0, keepdims=True))
    kn = k / jnp.maximum(ks, 1e-12)
    qs = jnp.sqrt(jnp.sum(q * q, axis=0, keepdims=True))
    qn = q / jnp.maximum(qs, 1e-12)

    sim = lax.dot_general(kn, qn, (((0,), (0,)), ((), ())),
                          precision=None, preferred_element_type=jnp.float32)
    sim = jnp.maximum(sim, 0.0)

    iota_m = lax.broadcasted_iota(jnp.int32, (n, n), 1)
    vhi = v.astype(jnp.bfloat16)
    vlo = (v - vhi.astype(jnp.float32)).astype(jnp.bfloat16)

    xc = jnp.zeros((c, n), jnp.float32)
    for kk in range(_K):
        mx = jnp.max(sim, axis=1, keepdims=True)
        am = jnp.min(jnp.where(sim == mx, iota_m, n), axis=1,
                     keepdims=True)
        sim = jnp.where(iota_m == am, -1.0, sim)
        pt = (iota_m == am).astype(jnp.bfloat16)
        g = nt_dot(vhi, pt, None) + nt_dot(vlo, pt, None)
        xc = xc + lax.dot_general(w2_ref[kk], g, (((1,), (0,)), ((), ())),
                                  precision=_HI,
                                  preferred_element_type=jnp.float32)
    xc = xc + b_ref[...]
    out_ref[0] = nt_dot(xc, wo_ref[...], _HI)


def kernel(x, Wq, Wk, Wv, Wo, conv_w, conv_b):
    b, c, h, w = x.shape
    n = h * w
    xf = x.reshape(b, c, n)
    w2 = conv_w.transpose(2, 0, 1)
    bias = conv_b.reshape(c, 1)

    full = lambda shp: pl.BlockSpec(shp, lambda i: tuple(0 for _ in shp))
    out = pl.pallas_call(
        _body,
        grid=(b,),
        in_specs=[
            pl.BlockSpec((1, c, n), lambda i: (i, 0, 0)),
            full((n, n)), full((n, n)), full((n, n)), full((n, n)),
            full((_K, c, c)),
            full((c, 1)),
        ],
        out_specs=pl.BlockSpec((1, c, n), lambda i: (i, 0, 0)),
        out_shape=jax.ShapeDtypeStruct((b, c, n), jnp.float32),
    )(xf, Wq, Wk, Wv, Wo, w2, bias)
    return out.reshape(b, c, h, w)


# speedup vs baseline: 13.5001x; 1.4709x over previous
import jax
import jax.numpy as jnp
from jax import lax
from jax.experimental import pallas as pl

_K = 8
_HI = lax.Precision.HIGHEST


def _body(x_ref, wq_ref, wk_ref, wv_ref, wo_ref, w2_ref, b_ref, out_ref):
    c, n = x_ref.shape[1], x_ref.shape[2]
    xb = x_ref[0]

    def nt_dot(a, b, prec):
        return lax.dot_general(a, b, (((1,), (1,)), ((), ())),
                               precision=prec,
                               preferred_element_type=jnp.float32)

    q = nt_dot(xb, wq_ref[...], None)
    k = nt_dot(xb, wk_ref[...], None)
    v = nt_dot(xb, wv_ref[...], None)

    ks = jnp.sqrt(jnp.sum(k * k, axis=0, keepdims=True))
    kn = k / jnp.maximum(ks, 1e-12)
    qs = jnp.sqrt(jnp.sum(q * q, axis=0, keepdims=True))
    qn = q / jnp.maximum(qs, 1e-12)

    sim = lax.dot_general(kn, qn, (((0,), (0,)), ((), ())),
                          precision=None, preferred_element_type=jnp.float32)
    sim = jnp.maximum(sim, 0.0)

    iota_m = lax.broadcasted_iota(jnp.int32, (n, n), 1)
    vhi = v.astype(jnp.bfloat16)
    vlo = (v - vhi.astype(jnp.float32)).astype(jnp.bfloat16)

    xc = jnp.zeros((c, n), jnp.float32)
    for kk in range(_K):
        mx = jnp.max(sim, axis=1, keepdims=True)
        am = jnp.min(jnp.where(sim == mx, iota_m, n), axis=1,
                     keepdims=True)
        sim = jnp.where(iota_m == am, -1.0, sim)
        pt = (iota_m == am).astype(jnp.bfloat16)
        g = nt_dot(vhi, pt, None) + nt_dot(vlo, pt, None)
        xc = xc + lax.dot_general(w2_ref[kk], g, (((1,), (0,)), ((), ())),
                                  precision=None,
                                  preferred_element_type=jnp.float32)
    xc = xc + b_ref[...]
    out_ref[0] = nt_dot(xc, wo_ref[...], None)


def kernel(x, Wq, Wk, Wv, Wo, conv_w, conv_b):
    b, c, h, w = x.shape
    n = h * w
    xf = x.reshape(b, c, n)
    w2 = conv_w.transpose(2, 0, 1)
    bias = conv_b.reshape(c, 1)

    full = lambda shp: pl.BlockSpec(shp, lambda i: tuple(0 for _ in shp))
    out = pl.pallas_call(
        _body,
        grid=(b,),
        in_specs=[
            pl.BlockSpec((1, c, n), lambda i: (i, 0, 0)),
            full((n, n)), full((n, n)), full((n, n)), full((n, n)),
            full((_K, c, c)),
            full((c, 1)),
        ],
        out_specs=pl.BlockSpec((1, c, n), lambda i: (i, 0, 0)),
        out_shape=jax.ShapeDtypeStruct((b, c, n), jnp.float32),
    )(xf, Wq, Wk, Wv, Wo, w2, bias)
    return out.reshape(b, c, h, w)
